# Initial kernel scaffold; baseline (speedup 1.0000x reference)
import jax
import jax.numpy as jnp
from jax.experimental import pallas as pl


def kernel(logit, mv):
    raise NotImplementedError("write your pallas kernel here")


# speedup vs baseline: 15.9458x; 15.9458x over previous
import jax
import jax.numpy as jnp
from jax import lax
from jax.experimental import pallas as pl
from jax.experimental.pallas import tpu as pltpu
from jax.experimental.pallas import tpu_sc as plsc

N = 1_000_000
K = 50
L = 16
NC = 2
NS = 16
NW = 25
ELS = N // NW
ROWS = ELS // L
UNR = 10
CAP = 256
NEG = -3.0e38
BIG = 3.0e38
BIGI = 2**30


def _scan_body(logit_hbm, mv_hbm, sums_hbm, colmax_hbm, candv_hbm, candi_hbm,
               l_v, m_v, stage_f, cv_st, ci_st):
    wid = lax.axis_index("s") * NC + lax.axis_index("c")

    @pl.when(wid < NW)
    def _():
        base = wid * ELS
        pltpu.sync_copy(logit_hbm.at[pl.ds(base, ELS)], l_v)
        pltpu.sync_copy(mv_hbm.at[pl.ds(base, ELS)], m_v)

        lanes = lax.iota(jnp.int32, L)
        zero = jnp.zeros((L,), jnp.float32)
        neg = jnp.full((L,), NEG, jnp.float32)

        def body(it, carry):
            acc = carry[0]
            cms = list(carry[1:])
            for u in range(UNR):
                off = (it * UNR + u) * L
                lv = l_v[pl.ds(off, L)]
                mv = m_v[pl.ds(off, L)]
                l1 = jnp.abs(lv - mv)
                rw = mv * 0.5 + 0.25
                cond = (lv < mv) | (l1 > 0.1)
                acc = acc + jnp.where(cond, rw * l1, zero)
                cms[u] = jnp.maximum(cms[u], mv)
            return tuple([acc] + cms)

        out = lax.fori_loop(0, ROWS // UNR, body,
                            tuple([zero] + [neg] * UNR))
        acc = out[0]
        cms = out[1:]

        def tbody(_, prev):
            m = neg
            for c in cms:
                m = jnp.maximum(m, jnp.where(c < prev, c, neg))
            mmax = jnp.max(m)
            return jnp.where(mmax > NEG, mmax, prev)

        t_loc = lax.fori_loop(0, K, tbody, jnp.float32(3.0e38))

        for j in range(CAP // L):
            cv_st[pl.ds(j * L, L)] = neg
            ci_st[pl.ds(j * L, L)] = jnp.zeros((L,), jnp.int32)

        def cbody(j, cnt):
            off = j * L
            mv = m_v[pl.ds(off, L)]
            mask = mv >= t_loc
            pos = cnt + jnp.cumsum(mask.astype(jnp.int32)) - 1
            pos = jnp.minimum(pos, CAP - 1)
            gidx = base + off + lanes
            plsc.store_scatter(cv_st, [pos], mv, mask=mask)
            plsc.store_scatter(ci_st, [pos], gidx, mask=mask)
            return cnt + plsc.all_reduce_population_count(mask)

        lax.fori_loop(0, ROWS, cbody, jnp.zeros((L,), jnp.int32))

        stage_f[...] = acc
        pltpu.sync_copy(stage_f, sums_hbm.at[pl.ds(wid * L, L)])
        cm = cms[0]
        for c in cms[1:]:
            cm = jnp.maximum(cm, c)
        stage_f[...] = cm
        pltpu.sync_copy(stage_f, colmax_hbm.at[pl.ds(wid * L, L)])
        pltpu.sync_copy(cv_st, candv_hbm.at[pl.ds(wid * CAP, CAP)])
        pltpu.sync_copy(ci_st, candi_hbm.at[pl.ds(wid * CAP, CAP)])


def _select_body(sums_hbm, colmax_hbm, candv_hbm, candi_hbm, logit_hbm,
                 out_hbm, sums_v, cm_v, cv_v, ci_v, compv, compi,
                 topv, topi, li_v, res_st, sem):
    wid = lax.axis_index("s") * NC + lax.axis_index("c")

    @pl.when(wid == 0)
    def _():
        pltpu.sync_copy(sums_hbm, sums_v)
        pltpu.sync_copy(colmax_hbm, cm_v)
        pltpu.sync_copy(candv_hbm, cv_v)
        pltpu.sync_copy(candi_hbm, ci_v)

        lanes = lax.iota(jnp.int32, L)
        neg = jnp.full((L,), NEG, jnp.float32)
        zero = jnp.zeros((L,), jnp.float32)

        def tgbody(_, prev):
            m = neg
            for j in range(NW):
                c = cm_v[pl.ds(j * L, L)]
                m = jnp.maximum(m, jnp.where(c < prev, c, neg))
            mmax = jnp.max(m)
            return jnp.where(mmax > NEG, mmax, prev)

        t_g = lax.fori_loop(0, K, tgbody, jnp.float32(3.0e38))

        for j in range(CAP // L):
            compv[pl.ds(j * L, L)] = neg
            compi[pl.ds(j * L, L)] = jnp.zeros((L,), jnp.int32)

        def fbody(j, cnt):
            off = j * L
            v = cv_v[pl.ds(off, L)]
            ix = ci_v[pl.ds(off, L)]
            mask = v >= t_g
            pos = cnt + jnp.cumsum(mask.astype(jnp.int32)) - 1
            pos = jnp.minimum(pos, CAP - 1)
            plsc.store_scatter(compv, [pos], v, mask=mask)
            plsc.store_scatter(compi, [pos], ix, mask=mask)
            return cnt + plsc.all_reduce_population_count(mask)

        lax.fori_loop(0, (NW * CAP) // L, fbody, jnp.zeros((L,), jnp.int32))

        for j in range(4):
            topv[pl.ds(j * L, L)] = zero
            topi[pl.ds(j * L, L)] = jnp.zeros((L,), jnp.int32)

        lane0 = lanes == 0

        def sbody(r, carry):
            vprev, iprev = carry
            m = neg
            for j in range(CAP // L):
                vj = compv[pl.ds(j * L, L)]
                ij = compi[pl.ds(j * L, L)]
                elig = (vj < vprev) | ((vj == vprev) & (ij > iprev))
                m = jnp.maximum(m, jnp.where(elig, vj, neg))
            mmax = jnp.max(m)
            imin = jnp.full((L,), BIGI, jnp.int32)
            for j in range(CAP // L):
                vj = compv[pl.ds(j * L, L)]
                ij = compi[pl.ds(j * L, L)]
                elig = (vj < vprev) | ((vj == vprev) & (ij > iprev))
                hit = elig & (vj == mmax)
                imin = jnp.minimum(imin, jnp.where(hit, ij, jnp.full((L,), BIGI, jnp.int32)))
            imn = jnp.min(imin)
            rsplat = jnp.full((L,), r, jnp.int32)
            plsc.store_scatter(topv, [rsplat], jnp.full((L,), mmax), mask=lane0)
            plsc.store_scatter(topi, [rsplat], jnp.full((L,), imn), mask=lane0)
            return mmax, imn

        lax.fori_loop(0, K, sbody, (jnp.float32(BIG), jnp.int32(-1)))

        pltpu.async_copy(logit_hbm.at[topi], li_v, sem).wait()

        corr = zero
        for j in range(4):
            rank = (lanes + j * L).astype(jnp.float32)
            lv = li_v[pl.ds(j * L, L)]
            mv = topv[pl.ds(j * L, L)]
            l1 = jnp.abs(lv - mv)
            rw = mv * 0.5 + 0.25
            cond = (lv < mv) | (l1 > 0.1)
            x = 1.0 - rank * (1.0 / K)
            mult = 2.0 * (x * x * x * 4.0 + 1.0)
            valid = cond & (rank < K)
            corr = corr + jnp.where(valid, rw * l1 * (mult - 1.0), zero)

        def gbody(i, carry):
            gs, cn = carry
            si = plsc.load_gather(li_v, [jnp.full((L,), i, jnp.int32)])
            for j in range(4):
                rank = lanes + j * L
                lj = li_v[pl.ds(j * L, L)]
                d = si - lj
                mask = (rank > i) & (rank < K) & (jnp.abs(d) < 0.05)
                gs = gs + jnp.where(mask, jnp.maximum(0.0, 0.1 - d), zero)
                cn = cn + jnp.where(mask, jnp.full((L,), 1.0), zero)
            return gs, cn

        gs, cn = lax.fori_loop(0, K, gbody, (zero, zero))

        ssum = zero
        for j in range(NW):
            ssum = ssum + sums_v[pl.ds(j * L, L)]

        total = (jnp.sum(ssum) + jnp.sum(corr)) * jnp.float32(1.0 / N)
        den = jnp.maximum(jnp.float32(1.0), jnp.sum(cn))
        gap_v = jnp.full((L,), jnp.sum(gs)) / jnp.full((L,), den)
        res_st[...] = jnp.full((L,), total) + gap_v
        pltpu.sync_copy(res_st, out_hbm)


def kernel(logit, mv):
    mesh = plsc.VectorSubcoreMesh(core_axis_name="c", subcore_axis_name="s")

    scan = pl.kernel(
        _scan_body,
        out_type=(
            jax.ShapeDtypeStruct((NW * L,), jnp.float32),
            jax.ShapeDtypeStruct((NW * L,), jnp.float32),
            jax.ShapeDtypeStruct((NW * CAP,), jnp.float32),
            jax.ShapeDtypeStruct((NW * CAP,), jnp.int32),
        ),
        mesh=mesh,
        compiler_params=pltpu.CompilerParams(needs_layout_passes=False),
        scratch_types=[
            pltpu.VMEM((ELS,), jnp.float32),
            pltpu.VMEM((ELS,), jnp.float32),
            pltpu.VMEM((L,), jnp.float32),
            pltpu.VMEM((CAP,), jnp.float32),
            pltpu.VMEM((CAP,), jnp.int32),
        ],
    )
    sums, colmax, candv, candi = scan(logit, mv)

    select = pl.kernel(
        _select_body,
        out_type=jax.ShapeDtypeStruct((L,), jnp.float32),
        mesh=mesh,
        compiler_params=pltpu.CompilerParams(needs_layout_passes=False),
        scratch_types=[
            pltpu.VMEM((NW * L,), jnp.float32),
            pltpu.VMEM((NW * L,), jnp.float32),
            pltpu.VMEM((NW * CAP,), jnp.float32),
            pltpu.VMEM((NW * CAP,), jnp.int32),
            pltpu.VMEM((CAP,), jnp.float32),
            pltpu.VMEM((CAP,), jnp.int32),
            pltpu.VMEM((4 * L,), jnp.float32),
            pltpu.VMEM((4 * L,), jnp.int32),
            pltpu.VMEM((4 * L,), jnp.float32),
            pltpu.VMEM((L,), jnp.float32),
            pltpu.SemaphoreType.DMA,
        ],
    )
    out = select(sums, colmax, candv, candi, logit)
    return out[0]


# speedup vs baseline: 23.0271x; 1.4441x over previous
import jax
import jax.numpy as jnp
from jax import lax
from jax.experimental import pallas as pl
from jax.experimental.pallas import tpu as pltpu
from jax.experimental.pallas import tpu_sc as plsc

N = 1_000_000
K = 50
L = 16
NC = 2
NS = 16
NW = 25
ELS = N // NW
ROWS = ELS // L
HROWS = ROWS // 2
UNR = 10
CAP = 256
CAPC = 128
NEG = -3.0e38
BIG = 3.0e38
BIGI = 2**30


def _scan_body(logit_hbm, mv_hbm, sums_hbm, colmax_hbm, cnts_hbm,
               candv_hbm, candi_hbm,
               l_v, m_v, stage_s, stage_c, stage_n, cv_st, ci_st,
               sem_a, sem_b, sem_o):
    wid = lax.axis_index("s") * NC + lax.axis_index("c")

    @pl.when(wid < NW)
    def _():
        base = wid * ELS
        half = ELS // 2
        d1 = pltpu.async_copy(logit_hbm.at[pl.ds(base, half)],
                              l_v.at[pl.ds(0, half)], sem_a)
        d2 = pltpu.async_copy(mv_hbm.at[pl.ds(base, half)],
                              m_v.at[pl.ds(0, half)], sem_a)
        d3 = pltpu.async_copy(logit_hbm.at[pl.ds(base + half, half)],
                              l_v.at[pl.ds(half, half)], sem_b)
        d4 = pltpu.async_copy(mv_hbm.at[pl.ds(base + half, half)],
                              m_v.at[pl.ds(half, half)], sem_b)

        lanes = lax.iota(jnp.int32, L)
        zero = jnp.zeros((L,), jnp.float32)
        neg = jnp.full((L,), NEG, jnp.float32)

        def make_body(row_base):
            def body(it, carry):
                acc = carry[0]
                cms = list(carry[1:])
                for u in range(UNR):
                    off = (row_base + it * UNR + u) * L
                    lv = l_v[pl.ds(off, L)]
                    mv = m_v[pl.ds(off, L)]
                    l1 = jnp.abs(lv - mv)
                    rw = mv * 0.5 + 0.25
                    cond = (lv < mv) | (l1 > 0.1)
                    acc = acc + jnp.where(cond, rw * l1, zero)
                    cms[u] = jnp.maximum(cms[u], mv)
                return tuple([acc] + cms)
            return body

        d1.wait()
        d2.wait()
        carry = lax.fori_loop(0, HROWS // UNR, make_body(0),
                              tuple([zero] + [neg] * UNR))
        d3.wait()
        d4.wait()
        carry = lax.fori_loop(0, HROWS // UNR, make_body(HROWS), carry)
        acc = carry[0]
        cms = carry[1:]

        def tbody(_, prev):
            m = neg
            for c in cms:
                m = jnp.maximum(m, jnp.where(c < prev, c, neg))
            mmax = jnp.max(m)
            return jnp.where(mmax > NEG, mmax, prev)

        t_loc = lax.fori_loop(0, K, tbody, jnp.float32(3.0e38))

        for j in range(CAP // L):
            cv_st[pl.ds(j * L, L)] = neg
            ci_st[pl.ds(j * L, L)] = jnp.zeros((L,), jnp.int32)

        def cbody(g, c):
            row0 = g * UNR
            m_or = m_v[pl.ds(row0 * L, L)] >= t_loc
            for u in range(1, UNR):
                m_or = m_or | (m_v[pl.ds((row0 + u) * L, L)] >= t_loc)
            anyhit = plsc.all_reduce_population_count(m_or)[0] > 0

            def hit(cc):
                for u in range(UNR):
                    off = (row0 + u) * L
                    mv = m_v[pl.ds(off, L)]
                    mask = mv >= t_loc
                    gidx = base + off + lanes
                    plsc.store_compressed(cv_st.at[pl.ds(cc, L)], mv, mask=mask)
                    plsc.store_compressed(ci_st.at[pl.ds(cc, L)], gidx, mask=mask)
                    pc = plsc.all_reduce_population_count(mask)[0]
                    cc = jnp.minimum(cc + pc, CAP - L)
                return cc

            return lax.cond(anyhit, hit, lambda cc: cc, c)

        cnt = lax.fori_loop(0, ROWS // UNR, cbody, jnp.int32(0))

        stage_s[...] = acc
        cm = cms[0]
        for c in cms[1:]:
            cm = jnp.maximum(cm, c)
        stage_c[...] = cm
        stage_n[...] = jnp.full((L,), cnt, jnp.int32)
        o1 = pltpu.async_copy(stage_s, sums_hbm.at[pl.ds(wid * L, L)], sem_o)
        o2 = pltpu.async_copy(stage_c, colmax_hbm.at[pl.ds(wid * L, L)], sem_o)
        o3 = pltpu.async_copy(stage_n, cnts_hbm.at[pl.ds(wid * L, L)], sem_o)
        o4 = pltpu.async_copy(cv_st, candv_hbm.at[pl.ds(wid * CAP, CAP)], sem_o)
        o5 = pltpu.async_copy(ci_st, candi_hbm.at[pl.ds(wid * CAP, CAP)], sem_o)
        o1.wait()
        o2.wait()
        o3.wait()
        o4.wait()
        o5.wait()


def _select_body(sums_hbm, colmax_hbm, cnts_hbm, candv_hbm, candi_hbm,
                 logit_hbm, out_hbm, sums_v, cm_v, cnt_v, cv_v, ci_v,
                 compv, compi, topv, topi, li_v, res_st, sem):
    wid = lax.axis_index("s") * NC + lax.axis_index("c")

    @pl.when(wid == 0)
    def _():
        i1 = pltpu.async_copy(sums_hbm, sums_v, sem)
        i2 = pltpu.async_copy(colmax_hbm, cm_v, sem)
        i3 = pltpu.async_copy(cnts_hbm, cnt_v, sem)
        i4 = pltpu.async_copy(candv_hbm, cv_v, sem)
        i5 = pltpu.async_copy(candi_hbm, ci_v, sem)
        i1.wait()
        i2.wait()
        i3.wait()
        i4.wait()
        i5.wait()

        lanes = lax.iota(jnp.int32, L)
        neg = jnp.full((L,), NEG, jnp.float32)
        zero = jnp.zeros((L,), jnp.float32)

        cmv = [cm_v[pl.ds(j * L, L)] for j in range(NW)]

        def tgbody(_, prev):
            m = neg
            for c in cmv:
                m = jnp.maximum(m, jnp.where(c < prev, c, neg))
            mmax = jnp.max(m)
            return jnp.where(mmax > NEG, mmax, prev)

        t_g = lax.fori_loop(0, K, tgbody, jnp.float32(3.0e38))

        for j in range(CAPC // L):
            compv[pl.ds(j * L, L)] = neg
            compi[pl.ds(j * L, L)] = jnp.zeros((L,), jnp.int32)

        c = jnp.int32(0)
        for w in range(NW):
            cw = cnt_v[pl.ds(w * L, L)][0]
            nv = lax.shift_right_logical(cw + (L - 1), 4)

            def fb(j, cc, w=w):
                v = cv_v[pl.ds(w * CAP + j * L, L)]
                ix = ci_v[pl.ds(w * CAP + j * L, L)]
                mask = v >= t_g
                plsc.store_compressed(compv.at[pl.ds(cc, L)], v, mask=mask)
                plsc.store_compressed(compi.at[pl.ds(cc, L)], ix, mask=mask)
                pc = plsc.all_reduce_population_count(mask)[0]
                return jnp.minimum(cc + pc, CAPC - L)

            c = lax.fori_loop(0, nv, fb, c)

        cvr = [compv[pl.ds(j * L, L)] for j in range(CAPC // L)]
        cir = [compi[pl.ds(j * L, L)] for j in range(CAPC // L)]

        for j in range(4):
            topv[pl.ds(j * L, L)] = zero
            topi[pl.ds(j * L, L)] = jnp.zeros((L,), jnp.int32)

        lane0 = lanes == 0
        bigi = jnp.full((L,), BIGI, jnp.int32)

        def sbody(r, carry):
            vprev, iprev = carry
            m = neg
            eligs = []
            for vj, ij in zip(cvr, cir):
                elig = (vj < vprev) | ((vj == vprev) & (ij > iprev))
                eligs.append(elig)
                m = jnp.maximum(m, jnp.where(elig, vj, neg))
            mmax = jnp.max(m)
            imin = bigi
            for vj, ij, elig in zip(cvr, cir, eligs):
                hit = elig & (vj == mmax)
                imin = jnp.minimum(imin, jnp.where(hit, ij, bigi))
            imn = jnp.min(imin)
            rsplat = jnp.full((L,), r, jnp.int32)
            plsc.store_scatter(topv, [rsplat], jnp.full((L,), mmax), mask=lane0)
            plsc.store_scatter(topi, [rsplat], jnp.full((L,), imn), mask=lane0)
            return mmax, imn

        lax.fori_loop(0, K, sbody, (jnp.float32(BIG), jnp.int32(-1)))

        pltpu.async_copy(logit_hbm.at[topi], li_v, sem).wait()

        corr = zero
        for j in range(4):
            rank = (lanes + j * L).astype(jnp.float32)
            lv = li_v[pl.ds(j * L, L)]
            mv = topv[pl.ds(j * L, L)]
            l1 = jnp.abs(lv - mv)
            rw = mv * 0.5 + 0.25
            cond = (lv < mv) | (l1 > 0.1)
            x = 1.0 - rank * (1.0 / K)
            mult = 2.0 * (x * x * x * 4.0 + 1.0)
            valid = cond & (rank < K)
            corr = corr + jnp.where(valid, rw * l1 * (mult - 1.0), zero)

        def gbody(i, carry):
            gs, cn = carry
            si = plsc.load_gather(li_v, [jnp.full((L,), i, jnp.int32)])
            for j in range(4):
                rank = lanes + j * L
                lj = li_v[pl.ds(j * L, L)]
                d = si - lj
                mask = (rank > i) & (rank < K) & (jnp.abs(d) < 0.05)
                gs = gs + jnp.where(mask, jnp.maximum(0.0, 0.1 - d), zero)
                cn = cn + jnp.where(mask, jnp.full((L,), 1.0), zero)
            return gs, cn

        gs, cn = lax.fori_loop(0, K, gbody, (zero, zero))

        ssum = zero
        for j in range(NW):
            ssum = ssum + sums_v[pl.ds(j * L, L)]

        total = (jnp.sum(ssum) + jnp.sum(corr)) * jnp.float32(1.0 / N)
        den = jnp.maximum(jnp.float32(1.0), jnp.sum(cn))
        gap_v = jnp.full((L,), jnp.sum(gs)) / jnp.full((L,), den)
        res_st[...] = jnp.full((L,), total) + gap_v
        pltpu.sync_copy(res_st, out_hbm)


def kernel(logit, mv):
    mesh = plsc.VectorSubcoreMesh(core_axis_name="c", subcore_axis_name="s")

    scan = pl.kernel(
        _scan_body,
        out_type=(
            jax.ShapeDtypeStruct((NW * L,), jnp.float32),
            jax.ShapeDtypeStruct((NW * L,), jnp.float32),
            jax.ShapeDtypeStruct((NW * L,), jnp.int32),
            jax.ShapeDtypeStruct((NW * CAP,), jnp.float32),
            jax.ShapeDtypeStruct((NW * CAP,), jnp.int32),
        ),
        mesh=mesh,
        compiler_params=pltpu.CompilerParams(needs_layout_passes=False),
        scratch_types=[
            pltpu.VMEM((ELS,), jnp.float32),
            pltpu.VMEM((ELS,), jnp.float32),
            pltpu.VMEM((L,), jnp.float32),
            pltpu.VMEM((L,), jnp.float32),
            pltpu.VMEM((L,), jnp.int32),
            pltpu.VMEM((CAP,), jnp.float32),
            pltpu.VMEM((CAP,), jnp.int32),
            pltpu.SemaphoreType.DMA,
            pltpu.SemaphoreType.DMA,
            pltpu.SemaphoreType.DMA,
        ],
    )
    sums, colmax, cnts, candv, candi = scan(logit, mv)

    select = pl.kernel(
        _select_body,
        out_type=jax.ShapeDtypeStruct((L,), jnp.float32),
        mesh=mesh,
        compiler_params=pltpu.CompilerParams(needs_layout_passes=False),
        scratch_types=[
            pltpu.VMEM((NW * L,), jnp.float32),
            pltpu.VMEM((NW * L,), jnp.float32),
            pltpu.VMEM((NW * L,), jnp.int32),
            pltpu.VMEM((NW * CAP,), jnp.float32),
            pltpu.VMEM((NW * CAP,), jnp.int32),
            pltpu.VMEM((CAPC,), jnp.float32),
            pltpu.VMEM((CAPC,), jnp.int32),
            pltpu.VMEM((4 * L,), jnp.float32),
            pltpu.VMEM((4 * L,), jnp.int32),
            pltpu.VMEM((4 * L,), jnp.float32),
            pltpu.VMEM((L,), jnp.float32),
            pltpu.SemaphoreType.DMA,
        ],
    )
    out = select(sums, colmax, cnts, candv, candi, logit)
    return out[0]
